# baseline (speedup 1.0000x reference)
import functools

import jax
import jax.numpy as jnp
from jax import lax
from jax.experimental import pallas as pl
from jax.experimental.pallas import tpu as pltpu
from jax.experimental.pallas import tpu_sc as plsc


def _sc_kernel(E, D, R, CHUNK):
  info = plsc.get_sparse_core_info()
  NC, NS, L = info.num_cores, info.num_subcores, info.num_lanes
  NW = NC * NS
  assert E % NW == 0
  e_per_w = E // NW
  assert e_per_w % CHUNK == 0 and CHUNK % L == 0
  n_chunks = e_per_w // CHUNK
  n_groups = CHUNK // L
  DW = D // 2

  mesh = plsc.VectorSubcoreMesh(core_axis_name="c", subcore_axis_name="s")

  @functools.partial(
      pl.kernel,
      mesh=mesh,
      compiler_params=pltpu.CompilerParams(
          needs_layout_passes=False, disable_bounds_checks=True,
          use_tc_tiling_on_sc=False),
      out_type=jax.ShapeDtypeStruct((E,), jnp.float32),
      scratch_types=[
          pltpu.VMEM((e_per_w,), jnp.int32),
          pltpu.VMEM((e_per_w,), jnp.int32),
          pltpu.VMEM((e_per_w,), jnp.int32),
          pltpu.VMEM((R, DW), jnp.int32),
          pltpu.VMEM((CHUNK, DW), jnp.int32),
          pltpu.VMEM((CHUNK, DW), jnp.int32),
          pltpu.VMEM((CHUNK, DW), jnp.int32),
          pltpu.VMEM((CHUNK, DW), jnp.int32),
          pltpu.VMEM((CHUNK, DW), jnp.int32),
          pltpu.VMEM((CHUNK, DW), jnp.int32),
          pltpu.VMEM((CHUNK, DW), jnp.int32),
          pltpu.VMEM((CHUNK, DW), jnp.int32),
          pltpu.VMEM((e_per_w,), jnp.float32),
          pltpu.SemaphoreType.DMA,
          pltpu.SemaphoreType.DMA,
          pltpu.SemaphoreType.DMA,
          pltpu.SemaphoreType.DMA,
          pltpu.SemaphoreType.DMA,
          pltpu.SemaphoreType.DMA,
          pltpu.SemaphoreType.DMA,
          pltpu.SemaphoreType.DMA,
      ],
  )
  def k(x_hbm, ei_hbm, et_hbm, w_hbm, out_hbm,
        sidx, didx, et_v, w_v, srows0, drows0, srows1, drows1,
        srows2, drows2, srows3, drows3, out_v,
        ss0, sd0, ss1, sd1, ss2, sd2, ss3, sd3):
    wid = lax.axis_index("s") * NC + lax.axis_index("c")
    base = wid * e_per_w

    pltpu.sync_copy(ei_hbm.at[pl.ds(base, e_per_w)], sidx)
    pltpu.sync_copy(ei_hbm.at[pl.ds(E + base, e_per_w)], didx)
    pltpu.sync_copy(et_hbm.at[pl.ds(base, e_per_w)], et_v)
    pltpu.sync_copy(w_hbm, w_v)

    lane = lax.broadcasted_iota(jnp.int32, (L,), 0)
    bufs = ((srows0, drows0, ss0, sd0), (srows1, drows1, ss1, sd1),
            (srows2, drows2, ss2, sd2), (srows3, drows3, ss3, sd3))

    def gather_pair(ch, b):
      sr, dr, ss, sd = bufs[b]
      cbase = ch * CHUNK
      pltpu.async_copy(x_hbm.at[sidx.at[pl.ds(cbase, CHUNK)]], sr, ss)
      pltpu.async_copy(x_hbm.at[didx.at[pl.ds(cbase, CHUNK)]], dr, sd)

    def wait_pair(ch, b):
      sr, dr, ss, sd = bufs[b]
      cbase = ch * CHUNK
      pltpu.make_async_copy(x_hbm.at[sidx.at[pl.ds(cbase, CHUNK)]], sr, ss).wait()
      pltpu.make_async_copy(x_hbm.at[didx.at[pl.ds(cbase, CHUNK)]], dr, sd).wait()

    def unpack2(iv):
      return plsc.unpack(plsc.bitcast(iv, jnp.bfloat16),
                         format=plsc.PackFormat.INTERLEAVED,
                         preferred_element_type=jnp.float32)

    def compute(ch, b):
      sr, dr, _, _ = bufs[b]
      cbase = ch * CHUNK

      zero = jnp.zeros((L,), jnp.int32)

      def group_body(g, _):
        eoff = cbase + g * L
        et16 = et_v[pl.ds(eoff, L)]
        ebase = (lane + g * L) * DW
        wbase = et16 * DW
        acc = jnp.zeros((L,), jnp.float32)
        for j in range(DW):
          dv = (lane + j) & (DW - 1)
          a = ebase + dv
          wa = wbase + dv
          si = plsc.load_gather(sr, [zero, a])
          ti = plsc.load_gather(dr, [zero, a])
          wi = plsc.load_gather(w_v, [zero, wa])
          sa, sb = unpack2(si)
          ta, tb = unpack2(ti)
          wva, wvb = unpack2(wi)
          acc = acc + sa * ta * wva + sb * tb * wvb
        out_v[pl.ds(eoff, L)] = 1.0 / (1.0 + jnp.exp(-acc))
        return 0

      lax.fori_loop(0, n_groups, group_body, 0)

    NBUF = 4
    assert n_chunks % NBUF == 1
    for i in range(NBUF - 1):
      gather_pair(i, i)

    def quad_body(chq, carry):
      for b in range(NBUF):
        ch = NBUF * chq + b
        wait_pair(ch, b)
        ch3 = ch + NBUF - 1

        @pl.when(ch3 < n_chunks)
        def _():
          gather_pair(ch3, (b + NBUF - 1) % NBUF)

        compute(ch, b)
      return carry

    lax.fori_loop(0, n_chunks // NBUF, quad_body, 0)
    wait_pair(n_chunks - 1, (n_chunks - 1) % NBUF)
    compute(n_chunks - 1, (n_chunks - 1) % NBUF)
    pltpu.sync_copy(out_v, out_hbm.at[pl.ds(base, e_per_w)])

  return k


def _pack_bf16_pairs(a):
  n, d = a.shape
  b = a.astype(jnp.bfloat16).reshape(n, d // 2, 2)
  return lax.bitcast_convert_type(b, jnp.int32)


def kernel(x, edge_index, edge_type, weight):
  E = edge_type.shape[0]
  D = x.shape[1]
  R = weight.shape[0]
  k = _sc_kernel(E, D, R, CHUNK=80)
  return k(_pack_bf16_pairs(x), edge_index.reshape(-1), edge_type,
           _pack_bf16_pairs(weight))


# speedup vs baseline: 1.0184x; 1.0184x over previous
import functools

import jax
import jax.numpy as jnp
from jax import lax
from jax.experimental import pallas as pl
from jax.experimental.pallas import tpu as pltpu
from jax.experimental.pallas import tpu_sc as plsc


def _sc_kernel(E, D, R, N, CHUNK):
  info = plsc.get_sparse_core_info()
  NC, NS, L = info.num_cores, info.num_subcores, info.num_lanes
  NW = NC * NS
  assert E % NW == 0
  e_per_w = E // NW
  assert e_per_w % CHUNK == 0 and CHUNK % L == 0
  n_chunks = e_per_w // CHUNK
  n_groups = CHUNK // L
  DW = D // 2

  mesh = plsc.VectorSubcoreMesh(core_axis_name="c", subcore_axis_name="s")

  @functools.partial(
      pl.kernel,
      mesh=mesh,
      compiler_params=pltpu.CompilerParams(
          needs_layout_passes=False, disable_bounds_checks=True,
          use_tc_tiling_on_sc=False),
      out_type=jax.ShapeDtypeStruct((E,), jnp.float32),
      scratch_types=[
          pltpu.VMEM((e_per_w,), jnp.int32),
          pltpu.VMEM((e_per_w,), jnp.int32),
          pltpu.VMEM((e_per_w,), jnp.int32),
          pltpu.VMEM((R, DW), jnp.int32),
          pltpu.VMEM((CHUNK, DW), jnp.int32),
          pltpu.VMEM((CHUNK, DW), jnp.int32),
          pltpu.VMEM((CHUNK, DW), jnp.int32),
          pltpu.VMEM((CHUNK, DW), jnp.int32),
          pltpu.VMEM((CHUNK, DW), jnp.int32),
          pltpu.VMEM((CHUNK, DW), jnp.int32),
          pltpu.VMEM((CHUNK, DW), jnp.int32),
          pltpu.VMEM((CHUNK, DW), jnp.int32),
          pltpu.VMEM((e_per_w,), jnp.float32),
          pltpu.VMEM_SHARED((N, DW), jnp.int32),
          pltpu.SemaphoreType.DMA,
          pltpu.SemaphoreType.DMA,
          pltpu.SemaphoreType.DMA,
          pltpu.SemaphoreType.DMA,
          pltpu.SemaphoreType.DMA,
          pltpu.SemaphoreType.DMA,
          pltpu.SemaphoreType.DMA,
          pltpu.SemaphoreType.DMA,
      ],
  )
  def k(x_hbm, ei_hbm, et_hbm, w_hbm, out_hbm,
        sidx, didx, et_v, w_v, srows0, drows0, srows1, drows1,
        srows2, drows2, srows3, drows3, out_v, x_sh,
        ss0, sd0, ss1, sd1, ss2, sd2, ss3, sd3):
    wid = lax.axis_index("s") * NC + lax.axis_index("c")
    base = wid * e_per_w

    @pl.when(lax.axis_index("s") == 0)
    def _():
      pltpu.sync_copy(x_hbm, x_sh)

    pltpu.sync_copy(ei_hbm.at[pl.ds(base, e_per_w)], sidx)
    pltpu.sync_copy(ei_hbm.at[pl.ds(E + base, e_per_w)], didx)
    pltpu.sync_copy(et_hbm.at[pl.ds(base, e_per_w)], et_v)
    pltpu.sync_copy(w_hbm, w_v)
    plsc.subcore_barrier()

    lane = lax.broadcasted_iota(jnp.int32, (L,), 0)
    bufs = ((srows0, drows0, ss0, sd0), (srows1, drows1, ss1, sd1),
            (srows2, drows2, ss2, sd2), (srows3, drows3, ss3, sd3))

    def gather_pair(ch, b):
      sr, dr, ss, sd = bufs[b]
      cbase = ch * CHUNK
      pltpu.async_copy(x_sh.at[sidx.at[pl.ds(cbase, CHUNK)]], sr, ss)
      pltpu.async_copy(x_sh.at[didx.at[pl.ds(cbase, CHUNK)]], dr, sd)

    def wait_pair(ch, b):
      sr, dr, ss, sd = bufs[b]
      cbase = ch * CHUNK
      pltpu.make_async_copy(x_sh.at[sidx.at[pl.ds(cbase, CHUNK)]], sr, ss).wait()
      pltpu.make_async_copy(x_sh.at[didx.at[pl.ds(cbase, CHUNK)]], dr, sd).wait()

    def unpack2(iv):
      return plsc.unpack(plsc.bitcast(iv, jnp.bfloat16),
                         format=plsc.PackFormat.INTERLEAVED,
                         preferred_element_type=jnp.float32)

    def compute(ch, b):
      sr, dr, _, _ = bufs[b]
      cbase = ch * CHUNK

      zero = jnp.zeros((L,), jnp.int32)

      def group_body(g, _):
        eoff = cbase + g * L
        et16 = et_v[pl.ds(eoff, L)]
        ebase = (lane + g * L) * DW
        wbase = et16 * DW
        acc = jnp.zeros((L,), jnp.float32)
        for j in range(DW):
          dv = (lane + j) & (DW - 1)
          a = ebase + dv
          wa = wbase + dv
          si = plsc.load_gather(sr, [zero, a])
          ti = plsc.load_gather(dr, [zero, a])
          wi = plsc.load_gather(w_v, [zero, wa])
          sa, sb = unpack2(si)
          ta, tb = unpack2(ti)
          wva, wvb = unpack2(wi)
          acc = acc + sa * ta * wva + sb * tb * wvb
        out_v[pl.ds(eoff, L)] = 1.0 / (1.0 + jnp.exp(-acc))
        return 0

      lax.fori_loop(0, n_groups, group_body, 0)

    NBUF = 4
    assert n_chunks % NBUF == 1
    for i in range(NBUF - 1):
      gather_pair(i, i)

    def quad_body(chq, carry):
      for b in range(NBUF):
        ch = NBUF * chq + b
        wait_pair(ch, b)
        ch3 = ch + NBUF - 1

        @pl.when(ch3 < n_chunks)
        def _():
          gather_pair(ch3, (b + NBUF - 1) % NBUF)

        compute(ch, b)
      return carry

    lax.fori_loop(0, n_chunks // NBUF, quad_body, 0)
    wait_pair(n_chunks - 1, (n_chunks - 1) % NBUF)
    compute(n_chunks - 1, (n_chunks - 1) % NBUF)
    pltpu.sync_copy(out_v, out_hbm.at[pl.ds(base, e_per_w)])

  return k


def _pack_bf16_pairs(a):
  n, d = a.shape
  b = a.astype(jnp.bfloat16).reshape(n, d // 2, 2)
  return lax.bitcast_convert_type(b, jnp.int32)


def kernel(x, edge_index, edge_type, weight):
  E = edge_type.shape[0]
  D = x.shape[1]
  R = weight.shape[0]
  k = _sc_kernel(E, D, R, x.shape[0], CHUNK=80)
  return k(_pack_bf16_pairs(x), edge_index.reshape(-1), edge_type,
           _pack_bf16_pairs(weight))


# speedup vs baseline: 1.0683x; 1.0490x over previous
import functools

import jax
import jax.numpy as jnp
from jax import lax
from jax.experimental import pallas as pl
from jax.experimental.pallas import tpu as pltpu
from jax.experimental.pallas import tpu_sc as plsc


def _sc_kernel(E, D, R, N, CHUNK):
  info = plsc.get_sparse_core_info()
  NC, NS, L = info.num_cores, info.num_subcores, info.num_lanes
  NW = NC * NS
  assert E % NW == 0
  e_per_w = E // NW
  assert e_per_w % CHUNK == 0 and CHUNK % L == 0
  n_chunks = e_per_w // CHUNK
  n_groups = CHUNK // L
  DW = D // 2

  mesh = plsc.VectorSubcoreMesh(core_axis_name="c", subcore_axis_name="s")

  @functools.partial(
      pl.kernel,
      mesh=mesh,
      compiler_params=pltpu.CompilerParams(
          needs_layout_passes=False, disable_bounds_checks=True,
          use_tc_tiling_on_sc=False),
      out_type=jax.ShapeDtypeStruct((E,), jnp.float32),
      scratch_types=[
          pltpu.VMEM((e_per_w,), jnp.int32),
          pltpu.VMEM((e_per_w,), jnp.int32),
          pltpu.VMEM((e_per_w,), jnp.int32),
          pltpu.VMEM((R, DW), jnp.int32),
          pltpu.VMEM((CHUNK, DW), jnp.int32),
          pltpu.VMEM((CHUNK, DW), jnp.int32),
          pltpu.VMEM((CHUNK, DW), jnp.int32),
          pltpu.VMEM((CHUNK, DW), jnp.int32),
          pltpu.VMEM((CHUNK, DW), jnp.int32),
          pltpu.VMEM((CHUNK, DW), jnp.int32),
          pltpu.VMEM((CHUNK, DW), jnp.int32),
          pltpu.VMEM((CHUNK, DW), jnp.int32),
          pltpu.VMEM((e_per_w,), jnp.float32),
          pltpu.VMEM_SHARED((N, DW), jnp.int32),
          pltpu.SemaphoreType.DMA,
          pltpu.SemaphoreType.DMA,
          pltpu.SemaphoreType.DMA,
          pltpu.SemaphoreType.DMA,
          pltpu.SemaphoreType.DMA,
          pltpu.SemaphoreType.DMA,
          pltpu.SemaphoreType.DMA,
          pltpu.SemaphoreType.DMA,
      ],
  )
  def k(x_hbm, ei_hbm, et_hbm, w_hbm, out_hbm,
        sidx, didx, et_v, w_v, srows0, drows0, srows1, drows1,
        srows2, drows2, srows3, drows3, out_v, x_sh,
        ss0, sd0, ss1, sd1, ss2, sd2, ss3, sd3):
    wid = lax.axis_index("s") * NC + lax.axis_index("c")
    base = wid * e_per_w

    @pl.when(lax.axis_index("s") == 0)
    def _():
      pltpu.sync_copy(x_hbm, x_sh)

    pltpu.sync_copy(ei_hbm.at[pl.ds(base, e_per_w)], sidx)
    pltpu.sync_copy(ei_hbm.at[pl.ds(E + base, e_per_w)], didx)
    pltpu.sync_copy(et_hbm.at[pl.ds(base, e_per_w)], et_v)
    pltpu.sync_copy(w_hbm, w_v)
    plsc.subcore_barrier()

    lane = lax.broadcasted_iota(jnp.int32, (L,), 0)
    bufs = ((srows0, drows0, ss0, sd0), (srows1, drows1, ss1, sd1),
            (srows2, drows2, ss2, sd2), (srows3, drows3, ss3, sd3))

    def gather_pair(ch, b):
      sr, dr, ss, sd = bufs[b]
      cbase = ch * CHUNK
      pltpu.async_copy(x_sh.at[sidx.at[pl.ds(cbase, CHUNK)]], sr, ss)
      pltpu.async_copy(x_sh.at[didx.at[pl.ds(cbase, CHUNK)]], dr, sd)

    def wait_pair(ch, b):
      sr, dr, ss, sd = bufs[b]
      cbase = ch * CHUNK
      pltpu.make_async_copy(x_sh.at[sidx.at[pl.ds(cbase, CHUNK)]], sr, ss).wait()
      pltpu.make_async_copy(x_sh.at[didx.at[pl.ds(cbase, CHUNK)]], dr, sd).wait()

    def unpack2(iv):
      return plsc.unpack(plsc.bitcast(iv, jnp.bfloat16),
                         format=plsc.PackFormat.INTERLEAVED,
                         preferred_element_type=jnp.float32)

    def compute(ch, b):
      sr, dr, _, _ = bufs[b]
      cbase = ch * CHUNK

      zero = jnp.zeros((L,), jnp.int32)

      def group_body(g, _):
        eoff = cbase + g * L
        et16 = et_v[pl.ds(eoff, L)]
        ebase = (lane + g * L) * DW
        wbase = et16 * DW
        acc = jnp.zeros((L,), jnp.float32)
        for j in range(DW):
          dv = (lane + j) & (DW - 1)
          a = ebase + dv
          wa = wbase + dv
          si = plsc.load_gather(sr, [zero, a])
          ti = plsc.load_gather(dr, [zero, a])
          wi = plsc.load_gather(w_v, [zero, wa])
          sa, sb = unpack2(si)
          ta, tb = unpack2(ti)
          wva, wvb = unpack2(wi)
          acc = acc + sa * ta * wva + sb * tb * wvb
        out_v[pl.ds(eoff, L)] = 1.0 / (1.0 + jnp.exp(-acc))
        return 0

      lax.fori_loop(0, n_groups, group_body, 0)

    assert n_chunks % 2 == 1
    gather_pair(0, 0)

    def pair_body(chp, carry):
      ch0 = 2 * chp
      wait_pair(ch0, 0)
      gather_pair(ch0 + 1, 1)
      compute(ch0, 0)
      wait_pair(ch0 + 1, 1)
      gather_pair(ch0 + 2, 0)
      compute(ch0 + 1, 1)
      return carry

    lax.fori_loop(0, (n_chunks - 1) // 2, pair_body, 0)
    wait_pair(n_chunks - 1, 0)
    compute(n_chunks - 1, 0)
    pltpu.sync_copy(out_v, out_hbm.at[pl.ds(base, e_per_w)])

  return k


def _pack_bf16_pairs(a):
  n, d = a.shape
  b = a.astype(jnp.bfloat16).reshape(n, d // 2, 2)
  return lax.bitcast_convert_type(b, jnp.int32)


def kernel(x, edge_index, edge_type, weight):
  E = edge_type.shape[0]
  D = x.shape[1]
  R = weight.shape[0]
  k = _sc_kernel(E, D, R, x.shape[0], CHUNK=80)
  return k(_pack_bf16_pairs(x), edge_index.reshape(-1), edge_type,
           _pack_bf16_pairs(weight))


# speedup vs baseline: 1.0796x; 1.0106x over previous
import functools

import jax
import jax.numpy as jnp
from jax import lax
from jax.experimental import pallas as pl
from jax.experimental.pallas import tpu as pltpu
from jax.experimental.pallas import tpu_sc as plsc


def _sc_kernel(E, D, R, CHUNK):
  info = plsc.get_sparse_core_info()
  NC, NS, L = info.num_cores, info.num_subcores, info.num_lanes
  NW = NC * NS
  assert E % NW == 0
  e_per_w = E // NW
  assert e_per_w % CHUNK == 0 and CHUNK % L == 0
  n_chunks = e_per_w // CHUNK
  n_groups = CHUNK // L
  DW = D // 2

  mesh = plsc.VectorSubcoreMesh(core_axis_name="c", subcore_axis_name="s")

  @functools.partial(
      pl.kernel,
      mesh=mesh,
      compiler_params=pltpu.CompilerParams(
          needs_layout_passes=False, disable_bounds_checks=True,
          use_tc_tiling_on_sc=False),
      out_type=jax.ShapeDtypeStruct((E,), jnp.float32),
      scratch_types=[
          pltpu.VMEM((e_per_w,), jnp.int32),
          pltpu.VMEM((e_per_w,), jnp.int32),
          pltpu.VMEM((e_per_w,), jnp.int32),
          pltpu.VMEM((R, DW), jnp.int32),
          pltpu.VMEM((CHUNK, DW), jnp.int32),
          pltpu.VMEM((CHUNK, DW), jnp.int32),
          pltpu.VMEM((CHUNK, DW), jnp.int32),
          pltpu.VMEM((CHUNK, DW), jnp.int32),
          pltpu.VMEM((e_per_w,), jnp.float32),
          pltpu.SemaphoreType.DMA,
          pltpu.SemaphoreType.DMA,
          pltpu.SemaphoreType.DMA,
          pltpu.SemaphoreType.DMA,
      ],
  )
  def k(x_hbm, ei_hbm, et_hbm, w_hbm, out_hbm,
        sidx, didx, et_v, w_v, srows0, drows0, srows1, drows1, out_v,
        ss0, sd0, ss1, sd1):
    wid = lax.axis_index("s") * NC + lax.axis_index("c")
    base = wid * e_per_w

    pltpu.sync_copy(ei_hbm.at[pl.ds(base, e_per_w)], sidx)
    pltpu.sync_copy(ei_hbm.at[pl.ds(E + base, e_per_w)], didx)
    pltpu.sync_copy(et_hbm.at[pl.ds(base, e_per_w)], et_v)
    pltpu.sync_copy(w_hbm, w_v)

    lane = lax.broadcasted_iota(jnp.int32, (L,), 0)
    bufs = ((srows0, drows0, ss0, sd0), (srows1, drows1, ss1, sd1))

    def gather_pair(ch, b):
      sr, dr, ss, sd = bufs[b]
      cbase = ch * CHUNK
      pltpu.async_copy(x_hbm.at[sidx.at[pl.ds(cbase, CHUNK)]], sr, ss)
      pltpu.async_copy(x_hbm.at[didx.at[pl.ds(cbase, CHUNK)]], dr, sd)

    def wait_pair(ch, b):
      sr, dr, ss, sd = bufs[b]
      cbase = ch * CHUNK
      pltpu.make_async_copy(x_hbm.at[sidx.at[pl.ds(cbase, CHUNK)]], sr, ss).wait()
      pltpu.make_async_copy(x_hbm.at[didx.at[pl.ds(cbase, CHUNK)]], dr, sd).wait()

    def unpack2(iv):
      return plsc.unpack(plsc.bitcast(iv, jnp.bfloat16),
                         format=plsc.PackFormat.INTERLEAVED,
                         preferred_element_type=jnp.float32)

    def compute(ch, b):
      sr, dr, _, _ = bufs[b]
      cbase = ch * CHUNK

      zero = jnp.zeros((L,), jnp.int32)

      def group_body(g, _):
        eoff = cbase + g * L
        et16 = et_v[pl.ds(eoff, L)]
        ebase = (lane + g * L) * DW
        wbase = et16 * DW
        acc = jnp.zeros((L,), jnp.float32)
        for j in range(DW):
          dv = (lane + j) & (DW - 1)
          a = ebase + dv
          wa = wbase + dv
          si = plsc.load_gather(sr, [zero, a])
          ti = plsc.load_gather(dr, [zero, a])
          wi = plsc.load_gather(w_v, [zero, wa])
          sa, sb = unpack2(si)
          ta, tb = unpack2(ti)
          wva, wvb = unpack2(wi)
          acc = acc + sa * ta * wva + sb * tb * wvb
        out_v[pl.ds(eoff, L)] = 1.0 / (1.0 + jnp.exp(-acc))
        return 0

      lax.fori_loop(0, n_groups, group_body, 0)

    assert n_chunks % 2 == 1
    gather_pair(0, 0)

    def pair_body(chp, carry):
      ch0 = 2 * chp
      wait_pair(ch0, 0)
      gather_pair(ch0 + 1, 1)
      compute(ch0, 0)
      wait_pair(ch0 + 1, 1)
      gather_pair(ch0 + 2, 0)
      compute(ch0 + 1, 1)
      return carry

    lax.fori_loop(0, (n_chunks - 1) // 2, pair_body, 0)
    wait_pair(n_chunks - 1, 0)
    compute(n_chunks - 1, 0)
    pltpu.sync_copy(out_v, out_hbm.at[pl.ds(base, e_per_w)])

  return k


def _pack_bf16_pairs(a):
  n, d = a.shape
  b = a.astype(jnp.bfloat16).reshape(n, d // 2, 2)
  return lax.bitcast_convert_type(b, jnp.int32)


def kernel(x, edge_index, edge_type, weight):
  E = edge_type.shape[0]
  D = x.shape[1]
  R = weight.shape[0]
  k = _sc_kernel(E, D, R, CHUNK=80)
  return k(_pack_bf16_pairs(x), edge_index.reshape(-1), edge_type,
           _pack_bf16_pairs(weight))
